# baseline (speedup 1.0000x reference)
import functools

import jax
import jax.numpy as jnp
from jax.experimental import pallas as pl
from jax.experimental.pallas import tpu as pltpu

_M_T = 512
_UNROLL = 4


def _tk_kernel(inp_ref, w1b_ref, b1_ref, w2b_ref, b2_ref, lut_ref, ind1_ref,
               mix_ref, out_ref, g_ref, d_ref):
    s = pl.program_id(0)
    t = pl.program_id(1)

    @pl.when((s == 0) & (t == 0))
    def _zero_out():
        out_ref[...] = jnp.zeros(out_ref.shape, out_ref.dtype)

    def gather_body(k, carry):
        for u in range(_UNROLL):
            m = k * _UNROLL + u
            for i in range(3):
                g_ref[i, m] = inp_ref[lut_ref[s * 3 + i, mix_ref[0, i, m]]]
        return carry

    jax.lax.fori_loop(0, _M_T // _UNROLL, gather_body, 0)

    x0 = g_ref[0].reshape(_M_T * 8, 256)
    x1 = g_ref[1].reshape(_M_T * 8, 256)
    x2 = g_ref[2].reshape(_M_T * 8, 256)
    z = (jnp.dot(x0, w1b_ref[0], preferred_element_type=jnp.float32)
         + jnp.dot(x1, w1b_ref[1], preferred_element_type=jnp.float32)
         + jnp.dot(x2, w1b_ref[2], preferred_element_type=jnp.float32))
    z = z + b1_ref[0:1, :]
    a = jnp.maximum(z, 0.0).astype(jnp.bfloat16)
    d = jnp.dot(a, w2b_ref[...], preferred_element_type=jnp.float32)
    d = d + b2_ref[0:1, :]
    d_ref[...] = d.reshape(_M_T, 8, 256)

    def scatter_body(k, carry):
        for u in range(_UNROLL):
            m = k * _UNROLL + u
            dval = d_ref[m]
            for i in range(3):
                c = ind1_ref[s * 3 + i, mix_ref[0, i, m]]
                out_ref[c] = jnp.maximum(out_ref[c], dval)
        return carry

    jax.lax.fori_loop(0, _M_T // _UNROLL, scatter_body, 0)


@functools.partial(jax.jit, static_argnums=())
def kernel(input_tensor, ind0, ind1, ind2, mix_ind, w1, b1, w2, b2):
    F_in, R, C = input_tensor.shape
    F_out = w2.shape[0]
    S = ind0.shape[0]
    M = mix_ind.shape[2]
    RF = R * F_in
    lanes = RF // 8

    inp_rows = jnp.transpose(input_tensor, (2, 1, 0)).reshape(C, 8, lanes)
    inp_rows = jnp.concatenate(
        [inp_rows, jnp.zeros((1, 8, lanes), inp_rows.dtype)], axis=0)
    inp_rows = inp_rows.astype(jnp.bfloat16)

    inds = (ind0, ind1, ind2)
    lut6 = jnp.stack([
        jnp.full((C,), C, jnp.int32).at[inds[i][s, :, 0]].set(
            inds[i][s, :, 1])
        for s in range(S) for i in range(3)])
    ind1_6 = jnp.stack([inds[i][s, :, 1]
                        for s in range(S) for i in range(3)])

    eye8 = jnp.eye(8, dtype=w1.dtype)
    w1b = jnp.stack([jnp.kron(eye8, w1[:, i * F_in:(i + 1) * F_in].T)
                     for i in range(3)]).astype(jnp.bfloat16)
    w2b = jnp.kron(eye8, w2.T).astype(jnp.bfloat16)
    b1b = jnp.broadcast_to(jnp.tile(b1, 8)[None, :], (8, 8 * w1.shape[0]))
    b2b = jnp.broadcast_to(jnp.tile(b2, 8)[None, :], (8, 8 * F_out))

    grid = (S, M // _M_T)
    out_rows = pl.pallas_call(
        _tk_kernel,
        grid=grid,
        in_specs=[
            pl.BlockSpec((C + 1, 8, lanes), lambda s, t: (0, 0, 0)),
            pl.BlockSpec((3, 256, 512), lambda s, t: (0, 0, 0)),
            pl.BlockSpec((8, 512), lambda s, t: (0, 0)),
            pl.BlockSpec((512, 256), lambda s, t: (0, 0)),
            pl.BlockSpec((8, 256), lambda s, t: (0, 0)),
            pl.BlockSpec((3 * S, C), lambda s, t: (0, 0),
                         memory_space=pltpu.SMEM),
            pl.BlockSpec((3 * S, ind0.shape[1]), lambda s, t: (0, 0),
                         memory_space=pltpu.SMEM),
            pl.BlockSpec((1, 3, _M_T), lambda s, t: (s, 0, t),
                         memory_space=pltpu.SMEM),
        ],
        out_specs=pl.BlockSpec((C, 8, lanes), lambda s, t: (0, 0, 0)),
        out_shape=jax.ShapeDtypeStruct((C, 8, lanes), jnp.float32),
        scratch_shapes=[
            pltpu.VMEM((3, _M_T, 8, lanes), jnp.bfloat16),
            pltpu.VMEM((_M_T, 8, lanes), jnp.float32),
        ],
        compiler_params=pltpu.CompilerParams(
            dimension_semantics=("arbitrary", "arbitrary")),
    )(inp_rows, w1b, b1b, w2b, b2b, lut6, ind1_6, mix_ind)

    return jnp.transpose(out_rows.reshape(C, R, F_out), (2, 1, 0))


# speedup vs baseline: 1.0094x; 1.0094x over previous
import functools

import jax
import jax.numpy as jnp
from jax.experimental import pallas as pl
from jax.experimental.pallas import tpu as pltpu

_M_T = 512
_UNROLL = 4


def _tk_kernel(inp_ref, w1b_ref, b1_ref, w2b_ref, b2_ref, i0_ref, i1_ref,
               i2_ref, mix_ref, out_ref, g_ref, d_ref, lut_ref):
    s = pl.program_id(0)
    t = pl.program_id(1)
    ind_refs = (i0_ref, i1_ref, i2_ref)
    n_sets = i0_ref.shape[0]
    n_ind = i0_ref.shape[2]
    c_tot = out_ref.shape[0]

    @pl.when((s == 0) & (t == 0))
    def _prologue():
        out_ref[...] = jnp.zeros(out_ref.shape, out_ref.dtype)

        def init_body(k, carry):
            for u in range(8):
                j = k * 8 + u
                for p in range(3 * n_sets):
                    lut_ref[p, j] = c_tot
            return carry

        jax.lax.fori_loop(0, c_tot // 8, init_body, 0)

        def build_body(k, carry):
            for s2 in range(n_sets):
                for i in range(3):
                    lut_ref[s2 * 3 + i, ind_refs[i][s2, 0, k]] = (
                        ind_refs[i][s2, 1, k])
            return carry

        jax.lax.fori_loop(0, n_ind, build_body, 0)

    def gather_body(k, carry):
        for u in range(_UNROLL):
            m = k * _UNROLL + u
            for i in range(3):
                g_ref[i, m] = inp_ref[lut_ref[s * 3 + i, mix_ref[0, i, m]]]
        return carry

    jax.lax.fori_loop(0, _M_T // _UNROLL, gather_body, 0)

    x0 = g_ref[0].reshape(_M_T * 8, 256)
    x1 = g_ref[1].reshape(_M_T * 8, 256)
    x2 = g_ref[2].reshape(_M_T * 8, 256)
    z = (jnp.dot(x0, w1b_ref[0], preferred_element_type=jnp.float32)
         + jnp.dot(x1, w1b_ref[1], preferred_element_type=jnp.float32)
         + jnp.dot(x2, w1b_ref[2], preferred_element_type=jnp.float32))
    z = z + b1_ref[0:1, :]
    a = jnp.maximum(z, 0.0).astype(jnp.bfloat16)
    d = jnp.dot(a, w2b_ref[...], preferred_element_type=jnp.float32)
    d = d + b2_ref[0:1, :]
    d_ref[...] = d.reshape(_M_T, 8, 256)

    def scatter_body(k, carry):
        for u in range(_UNROLL):
            m = k * _UNROLL + u
            dval = d_ref[m]
            for i in range(3):
                c = ind_refs[i][s, 1, mix_ref[0, i, m]]
                out_ref[c] = jnp.maximum(out_ref[c], dval)
        return carry

    jax.lax.fori_loop(0, _M_T // _UNROLL, scatter_body, 0)


@functools.partial(jax.jit, static_argnums=())
def kernel(input_tensor, ind0, ind1, ind2, mix_ind, w1, b1, w2, b2):
    F_in, R, C = input_tensor.shape
    F_out = w2.shape[0]
    S = ind0.shape[0]
    M = mix_ind.shape[2]
    RF = R * F_in
    lanes = RF // 8

    inp_rows = jnp.transpose(input_tensor, (2, 1, 0)).reshape(C, 8, lanes)
    inp_rows = jnp.concatenate(
        [inp_rows, jnp.zeros((1, 8, lanes), inp_rows.dtype)], axis=0)
    inp_rows = inp_rows.astype(jnp.bfloat16)

    ind0t = jnp.transpose(ind0, (0, 2, 1))
    ind1t = jnp.transpose(ind1, (0, 2, 1))
    ind2t = jnp.transpose(ind2, (0, 2, 1))

    eye8 = jnp.eye(8, dtype=w1.dtype)
    w1b = jnp.stack([jnp.kron(eye8, w1[:, i * F_in:(i + 1) * F_in].T)
                     for i in range(3)]).astype(jnp.bfloat16)
    w2b = jnp.kron(eye8, w2.T).astype(jnp.bfloat16)
    b1b = jnp.broadcast_to(jnp.tile(b1, 8)[None, :], (8, 8 * w1.shape[0]))
    b2b = jnp.broadcast_to(jnp.tile(b2, 8)[None, :], (8, 8 * F_out))

    grid = (S, M // _M_T)
    out_rows = pl.pallas_call(
        _tk_kernel,
        grid=grid,
        in_specs=[
            pl.BlockSpec((C + 1, 8, lanes), lambda s, t: (0, 0, 0)),
            pl.BlockSpec((3, 256, 512), lambda s, t: (0, 0, 0)),
            pl.BlockSpec((8, 512), lambda s, t: (0, 0)),
            pl.BlockSpec((512, 256), lambda s, t: (0, 0)),
            pl.BlockSpec((8, 256), lambda s, t: (0, 0)),
            pl.BlockSpec(ind0t.shape, lambda s, t: (0, 0, 0),
                         memory_space=pltpu.SMEM),
            pl.BlockSpec(ind0t.shape, lambda s, t: (0, 0, 0),
                         memory_space=pltpu.SMEM),
            pl.BlockSpec(ind0t.shape, lambda s, t: (0, 0, 0),
                         memory_space=pltpu.SMEM),
            pl.BlockSpec((1, 3, _M_T), lambda s, t: (s, 0, t),
                         memory_space=pltpu.SMEM),
        ],
        out_specs=pl.BlockSpec((C, 8, lanes), lambda s, t: (0, 0, 0)),
        out_shape=jax.ShapeDtypeStruct((C, 8, lanes), jnp.float32),
        scratch_shapes=[
            pltpu.VMEM((3, _M_T, 8, lanes), jnp.bfloat16),
            pltpu.VMEM((_M_T, 8, lanes), jnp.float32),
            pltpu.SMEM((3 * S, C), jnp.int32),
        ],
        compiler_params=pltpu.CompilerParams(
            dimension_semantics=("arbitrary", "arbitrary")),
    )(inp_rows, w1b, b1b, w2b, b2b, ind0t, ind1t, ind2t, mix_ind)

    return jnp.transpose(out_rows.reshape(C, R, F_out), (2, 1, 0))


# speedup vs baseline: 1.0314x; 1.0218x over previous
import functools

import jax
import jax.numpy as jnp
from jax.experimental import pallas as pl
from jax.experimental.pallas import tpu as pltpu

_M_T = 512
_UNROLL = 8


def _tk_kernel(inp_ref, w1b_ref, b1_ref, w2b_ref, b2_ref, i0_ref, i1_ref,
               i2_ref, mix_ref, out_ref, g_ref, d_ref, lut_ref):
    s = pl.program_id(0)
    t = pl.program_id(1)
    ind_refs = (i0_ref, i1_ref, i2_ref)
    n_sets = i0_ref.shape[0]
    n_ind = i0_ref.shape[2]
    c_tot = out_ref.shape[0]

    @pl.when((s == 0) & (t == 0))
    def _prologue():
        out_ref[...] = jnp.zeros(out_ref.shape, out_ref.dtype)

        def init_body(k, carry):
            for u in range(8):
                j = k * 8 + u
                for p in range(3 * n_sets):
                    lut_ref[p, j] = c_tot
            return carry

        jax.lax.fori_loop(0, c_tot // 8, init_body, 0)

        def build_body(k, carry):
            for s2 in range(n_sets):
                for i in range(3):
                    lut_ref[s2 * 3 + i, ind_refs[i][s2, 0, k]] = (
                        ind_refs[i][s2, 1, k])
            return carry

        jax.lax.fori_loop(0, n_ind, build_body, 0)

    def gather_body(k, carry):
        for u in range(_UNROLL):
            m = k * _UNROLL + u
            for i in range(3):
                g_ref[i, m] = inp_ref[lut_ref[s * 3 + i, mix_ref[0, i, m]]]
        return carry

    jax.lax.fori_loop(0, _M_T // _UNROLL, gather_body, 0)

    x0 = g_ref[0].reshape(_M_T * 8, 256)
    x1 = g_ref[1].reshape(_M_T * 8, 256)
    x2 = g_ref[2].reshape(_M_T * 8, 256)
    z = (jnp.dot(x0, w1b_ref[0], preferred_element_type=jnp.float32)
         + jnp.dot(x1, w1b_ref[1], preferred_element_type=jnp.float32)
         + jnp.dot(x2, w1b_ref[2], preferred_element_type=jnp.float32))
    z = z + b1_ref[0:1, :]
    a = jnp.maximum(z, 0.0).astype(jnp.bfloat16)
    d = jnp.dot(a, w2b_ref[...], preferred_element_type=jnp.float32)
    d = d + b2_ref[0:1, :]
    d_ref[...] = d.reshape(_M_T, 8, 256)

    def scatter_body(k, carry):
        for u in range(_UNROLL):
            m = k * _UNROLL + u
            dval = d_ref[m]
            for i in range(3):
                c = ind_refs[i][s, 1, mix_ref[0, i, m]]
                out_ref[c] = jnp.maximum(out_ref[c], dval)
        return carry

    jax.lax.fori_loop(0, _M_T // _UNROLL, scatter_body, 0)


@functools.partial(jax.jit, static_argnums=())
def kernel(input_tensor, ind0, ind1, ind2, mix_ind, w1, b1, w2, b2):
    F_in, R, C = input_tensor.shape
    F_out = w2.shape[0]
    S = ind0.shape[0]
    M = mix_ind.shape[2]
    RF = R * F_in
    lanes = RF // 8

    inp_rows = jnp.transpose(input_tensor, (2, 1, 0)).reshape(C, 8, lanes)
    inp_rows = jnp.concatenate(
        [inp_rows, jnp.zeros((1, 8, lanes), inp_rows.dtype)], axis=0)
    inp_rows = inp_rows.astype(jnp.bfloat16)

    ind0t = jnp.transpose(ind0, (0, 2, 1))
    ind1t = jnp.transpose(ind1, (0, 2, 1))
    ind2t = jnp.transpose(ind2, (0, 2, 1))

    eye8 = jnp.eye(8, dtype=w1.dtype)
    w1b = jnp.stack([jnp.kron(eye8, w1[:, i * F_in:(i + 1) * F_in].T)
                     for i in range(3)]).astype(jnp.bfloat16)
    w2b = jnp.kron(eye8, w2.T).astype(jnp.bfloat16)
    b1b = jnp.broadcast_to(jnp.tile(b1, 8)[None, :], (8, 8 * w1.shape[0]))
    b2b = jnp.broadcast_to(jnp.tile(b2, 8)[None, :], (8, 8 * F_out))

    grid = (S, M // _M_T)
    out_rows = pl.pallas_call(
        _tk_kernel,
        grid=grid,
        in_specs=[
            pl.BlockSpec((C + 1, 8, lanes), lambda s, t: (0, 0, 0)),
            pl.BlockSpec((3, 256, 512), lambda s, t: (0, 0, 0)),
            pl.BlockSpec((8, 512), lambda s, t: (0, 0)),
            pl.BlockSpec((512, 256), lambda s, t: (0, 0)),
            pl.BlockSpec((8, 256), lambda s, t: (0, 0)),
            pl.BlockSpec(ind0t.shape, lambda s, t: (0, 0, 0),
                         memory_space=pltpu.SMEM),
            pl.BlockSpec(ind0t.shape, lambda s, t: (0, 0, 0),
                         memory_space=pltpu.SMEM),
            pl.BlockSpec(ind0t.shape, lambda s, t: (0, 0, 0),
                         memory_space=pltpu.SMEM),
            pl.BlockSpec((1, 3, _M_T), lambda s, t: (s, 0, t),
                         memory_space=pltpu.SMEM),
        ],
        out_specs=pl.BlockSpec((C, 8, lanes), lambda s, t: (0, 0, 0)),
        out_shape=jax.ShapeDtypeStruct((C, 8, lanes), jnp.float32),
        scratch_shapes=[
            pltpu.VMEM((3, _M_T, 8, lanes), jnp.bfloat16),
            pltpu.VMEM((_M_T, 8, lanes), jnp.float32),
            pltpu.SMEM((3 * S, C), jnp.int32),
        ],
        compiler_params=pltpu.CompilerParams(
            dimension_semantics=("arbitrary", "arbitrary")),
    )(inp_rows, w1b, b1b, w2b, b2b, ind0t, ind1t, ind2t, mix_ind)

    return jnp.transpose(out_rows.reshape(C, R, F_out), (2, 1, 0))


# speedup vs baseline: 1.2425x; 1.2046x over previous
import functools

import jax
import jax.numpy as jnp
from jax import lax
from jax.experimental import pallas as pl
from jax.experimental.pallas import tpu as pltpu
from jax.experimental.pallas import tpu_sc as plsc

_M_T = 512
_UNROLL = 8
_QW = 4


def _sc_index_kernel(lut_hbm, ind1_hbm, mix_hbm, src_hbm, dst_hbm,
                     lut_v, ind1_v, mix_v, src_v, dst_v):
    n_pairs = lut_hbm.shape[0]
    m_tot = mix_hbm.shape[1]
    chunk = m_tot // _QW
    nc = plsc.get_sparse_core_info().num_cores
    wid = lax.axis_index("s") * nc + lax.axis_index("c")

    @pl.when(wid < n_pairs * _QW)
    def _work():
        p = wid // _QW
        q = wid % _QW
        pltpu.sync_copy(lut_hbm.at[p], lut_v)
        pltpu.sync_copy(ind1_hbm.at[p], ind1_v)
        pltpu.sync_copy(mix_hbm.at[p, pl.ds(q * chunk, chunk)], mix_v)

        def body(j, carry):
            idx = mix_v[pl.ds(j * 16, 16)]
            src_v[pl.ds(j * 16, 16)] = plsc.load_gather(lut_v, [idx])
            dst_v[pl.ds(j * 16, 16)] = plsc.load_gather(ind1_v, [idx])
            return carry

        lax.fori_loop(0, chunk // 16, body, 0)
        pltpu.sync_copy(src_v, src_hbm.at[p, pl.ds(q * chunk, chunk)])
        pltpu.sync_copy(dst_v, dst_hbm.at[p, pl.ds(q * chunk, chunk)])


def _tk_kernel(inp_ref, w1b_ref, b1_ref, w2b_ref, b2_ref, src_ref, dst_ref,
               out_ref, g_ref, d_ref):
    s = pl.program_id(0)
    t = pl.program_id(1)

    @pl.when((s == 0) & (t == 0))
    def _zero_out():
        out_ref[...] = jnp.zeros(out_ref.shape, out_ref.dtype)

    def gather_body(k, carry):
        for u in range(_UNROLL):
            m = k * _UNROLL + u
            for i in range(3):
                g_ref[i, m] = inp_ref[src_ref[0, i, m]]
        return carry

    lax.fori_loop(0, _M_T // _UNROLL, gather_body, 0)

    x0 = g_ref[0].reshape(_M_T * 8, 256)
    x1 = g_ref[1].reshape(_M_T * 8, 256)
    x2 = g_ref[2].reshape(_M_T * 8, 256)
    z = (jnp.dot(x0, w1b_ref[0], preferred_element_type=jnp.float32)
         + jnp.dot(x1, w1b_ref[1], preferred_element_type=jnp.float32)
         + jnp.dot(x2, w1b_ref[2], preferred_element_type=jnp.float32))
    z = z + b1_ref[0:1, :]
    a = jnp.maximum(z, 0.0).astype(jnp.bfloat16)
    d = jnp.dot(a, w2b_ref[...], preferred_element_type=jnp.float32)
    d = d + b2_ref[0:1, :]
    d_ref[...] = d.reshape(_M_T, 8, 256)

    def scatter_body(k, carry):
        for u in range(_UNROLL):
            m = k * _UNROLL + u
            dval = d_ref[m]
            for i in range(3):
                c = dst_ref[0, i, m]
                out_ref[c] = jnp.maximum(out_ref[c], dval)
        return carry

    lax.fori_loop(0, _M_T // _UNROLL, scatter_body, 0)


@functools.partial(jax.jit, static_argnums=())
def kernel(input_tensor, ind0, ind1, ind2, mix_ind, w1, b1, w2, b2):
    F_in, R, C = input_tensor.shape
    F_out = w2.shape[0]
    S = ind0.shape[0]
    M = mix_ind.shape[2]
    RF = R * F_in
    lanes = RF // 8

    inp_rows = jnp.transpose(input_tensor, (2, 1, 0)).reshape(C, 8, lanes)
    inp_rows = jnp.concatenate(
        [inp_rows, jnp.zeros((1, 8, lanes), inp_rows.dtype)], axis=0)
    inp_rows = inp_rows.astype(jnp.bfloat16)

    inds = (ind0, ind1, ind2)
    lut6 = jnp.stack([
        jnp.full((C,), C, jnp.int32).at[inds[i][s, :, 0]].set(
            inds[i][s, :, 1])
        for s in range(S) for i in range(3)])
    ind1_6 = jnp.stack([inds[i][s, :, 1]
                        for s in range(S) for i in range(3)])
    mix6 = mix_ind.reshape(3 * S, M)

    n_ind = ind0.shape[1]
    sc_mesh = plsc.VectorSubcoreMesh(core_axis_name="c", subcore_axis_name="s")
    src6, dst6 = pl.kernel(
        _sc_index_kernel,
        out_type=(jax.ShapeDtypeStruct((3 * S, M), jnp.int32),
                  jax.ShapeDtypeStruct((3 * S, M), jnp.int32)),
        mesh=sc_mesh,
        scratch_types=[
            pltpu.VMEM((C,), jnp.int32),
            pltpu.VMEM((n_ind,), jnp.int32),
            pltpu.VMEM((M // _QW,), jnp.int32),
            pltpu.VMEM((M // _QW,), jnp.int32),
            pltpu.VMEM((M // _QW,), jnp.int32),
        ],
        compiler_params=pltpu.CompilerParams(needs_layout_passes=False),
    )(lut6, ind1_6, mix6)
    src_all = src6.reshape(S, 3, M)
    dst_all = dst6.reshape(S, 3, M)

    eye8 = jnp.eye(8, dtype=w1.dtype)
    w1b = jnp.stack([jnp.kron(eye8, w1[:, i * F_in:(i + 1) * F_in].T)
                     for i in range(3)]).astype(jnp.bfloat16)
    w2b = jnp.kron(eye8, w2.T).astype(jnp.bfloat16)
    b1b = jnp.broadcast_to(jnp.tile(b1, 8)[None, :], (8, 8 * w1.shape[0]))
    b2b = jnp.broadcast_to(jnp.tile(b2, 8)[None, :], (8, 8 * F_out))

    grid = (S, M // _M_T)
    out_rows = pl.pallas_call(
        _tk_kernel,
        grid=grid,
        in_specs=[
            pl.BlockSpec((C + 1, 8, lanes), lambda s, t: (0, 0, 0)),
            pl.BlockSpec((3, 256, 512), lambda s, t: (0, 0, 0)),
            pl.BlockSpec((8, 512), lambda s, t: (0, 0)),
            pl.BlockSpec((512, 256), lambda s, t: (0, 0)),
            pl.BlockSpec((8, 256), lambda s, t: (0, 0)),
            pl.BlockSpec((1, 3, _M_T), lambda s, t: (s, 0, t),
                         memory_space=pltpu.SMEM),
            pl.BlockSpec((1, 3, _M_T), lambda s, t: (s, 0, t),
                         memory_space=pltpu.SMEM),
        ],
        out_specs=pl.BlockSpec((C, 8, lanes), lambda s, t: (0, 0, 0)),
        out_shape=jax.ShapeDtypeStruct((C, 8, lanes), jnp.float32),
        scratch_shapes=[
            pltpu.VMEM((3, _M_T, 8, lanes), jnp.bfloat16),
            pltpu.VMEM((_M_T, 8, lanes), jnp.float32),
        ],
        compiler_params=pltpu.CompilerParams(
            dimension_semantics=("arbitrary", "arbitrary")),
    )(inp_rows, w1b, b1b, w2b, b2b, src_all, dst_all)

    return jnp.transpose(out_rows.reshape(C, R, F_out), (2, 1, 0))


# speedup vs baseline: 1.2523x; 1.0079x over previous
import functools

import jax
import jax.numpy as jnp
from jax import lax
from jax.experimental import pallas as pl
from jax.experimental.pallas import tpu as pltpu
from jax.experimental.pallas import tpu_sc as plsc

_M_T = 1024
_UNROLL = 16
_QW = 4


def _sc_index_kernel(lut_hbm, ind1_hbm, mix_hbm, src_hbm, dst_hbm,
                     lut_v, ind1_v, mix_v, src_v, dst_v):
    n_pairs = lut_hbm.shape[0]
    m_tot = mix_hbm.shape[1]
    chunk = m_tot // _QW
    nc = plsc.get_sparse_core_info().num_cores
    wid = lax.axis_index("s") * nc + lax.axis_index("c")

    @pl.when(wid < n_pairs * _QW)
    def _work():
        p = wid // _QW
        q = wid % _QW
        pltpu.sync_copy(lut_hbm.at[p], lut_v)
        pltpu.sync_copy(ind1_hbm.at[p], ind1_v)
        pltpu.sync_copy(mix_hbm.at[p, pl.ds(q * chunk, chunk)], mix_v)

        def body(j, carry):
            idx = mix_v[pl.ds(j * 16, 16)]
            src_v[pl.ds(j * 16, 16)] = plsc.load_gather(lut_v, [idx])
            dst_v[pl.ds(j * 16, 16)] = plsc.load_gather(ind1_v, [idx])
            return carry

        lax.fori_loop(0, chunk // 16, body, 0)
        pltpu.sync_copy(src_v, src_hbm.at[p, pl.ds(q * chunk, chunk)])
        pltpu.sync_copy(dst_v, dst_hbm.at[p, pl.ds(q * chunk, chunk)])


def _tk_kernel(inp_ref, w1b_ref, b1_ref, w2b_ref, b2_ref, src_ref, dst_ref,
               out_ref, g_ref, d_ref):
    s = pl.program_id(0)
    t = pl.program_id(1)

    @pl.when((s == 0) & (t == 0))
    def _zero_out():
        out_ref[...] = jnp.zeros(out_ref.shape, out_ref.dtype)

    def gather_body(k, carry):
        for u in range(_UNROLL):
            m = k * _UNROLL + u
            for i in range(3):
                g_ref[i, m] = inp_ref[src_ref[0, i, m]]
        return carry

    lax.fori_loop(0, _M_T // _UNROLL, gather_body, 0)

    x0 = g_ref[0].reshape(_M_T * 8, 256)
    x1 = g_ref[1].reshape(_M_T * 8, 256)
    x2 = g_ref[2].reshape(_M_T * 8, 256)
    z = (jnp.dot(x0, w1b_ref[0], preferred_element_type=jnp.float32)
         + jnp.dot(x1, w1b_ref[1], preferred_element_type=jnp.float32)
         + jnp.dot(x2, w1b_ref[2], preferred_element_type=jnp.float32))
    z = z + b1_ref[0:1, :]
    a = jnp.maximum(z, 0.0).astype(jnp.bfloat16)
    d = jnp.dot(a, w2b_ref[...], preferred_element_type=jnp.float32)
    d = d + b2_ref[0:1, :]
    d_ref[...] = d.reshape(_M_T, 8, 256)

    def scatter_body(k, carry):
        for u in range(_UNROLL):
            m = k * _UNROLL + u
            dval = d_ref[m]
            for i in range(3):
                c = dst_ref[0, i, m]
                out_ref[c] = jnp.maximum(out_ref[c], dval)
        return carry

    lax.fori_loop(0, _M_T // _UNROLL, scatter_body, 0)


@functools.partial(jax.jit, static_argnums=())
def kernel(input_tensor, ind0, ind1, ind2, mix_ind, w1, b1, w2, b2):
    F_in, R, C = input_tensor.shape
    F_out = w2.shape[0]
    S = ind0.shape[0]
    M = mix_ind.shape[2]
    RF = R * F_in
    lanes = RF // 8

    inp_rows = jnp.transpose(input_tensor, (2, 1, 0)).reshape(C, 8, lanes)
    inp_rows = jnp.concatenate(
        [inp_rows, jnp.zeros((1, 8, lanes), inp_rows.dtype)], axis=0)
    inp_rows = inp_rows.astype(jnp.bfloat16)

    inds = (ind0, ind1, ind2)
    lut6 = jnp.stack([
        jnp.full((C,), C, jnp.int32).at[inds[i][s, :, 0]].set(
            inds[i][s, :, 1])
        for s in range(S) for i in range(3)])
    ind1_6 = jnp.stack([inds[i][s, :, 1]
                        for s in range(S) for i in range(3)])
    mix6 = mix_ind.reshape(3 * S, M)

    n_ind = ind0.shape[1]
    sc_mesh = plsc.VectorSubcoreMesh(core_axis_name="c", subcore_axis_name="s")
    src6, dst6 = pl.kernel(
        _sc_index_kernel,
        out_type=(jax.ShapeDtypeStruct((3 * S, M), jnp.int32),
                  jax.ShapeDtypeStruct((3 * S, M), jnp.int32)),
        mesh=sc_mesh,
        scratch_types=[
            pltpu.VMEM((C,), jnp.int32),
            pltpu.VMEM((n_ind,), jnp.int32),
            pltpu.VMEM((M // _QW,), jnp.int32),
            pltpu.VMEM((M // _QW,), jnp.int32),
            pltpu.VMEM((M // _QW,), jnp.int32),
        ],
        compiler_params=pltpu.CompilerParams(needs_layout_passes=False),
    )(lut6, ind1_6, mix6)
    src_all = src6.reshape(S, 3, M)
    dst_all = dst6.reshape(S, 3, M)

    eye8 = jnp.eye(8, dtype=w1.dtype)
    w1b = jnp.stack([jnp.kron(eye8, w1[:, i * F_in:(i + 1) * F_in].T)
                     for i in range(3)]).astype(jnp.bfloat16)
    w2b = jnp.kron(eye8, w2.T).astype(jnp.bfloat16)
    b1b = jnp.broadcast_to(jnp.tile(b1, 8)[None, :], (8, 8 * w1.shape[0]))
    b2b = jnp.broadcast_to(jnp.tile(b2, 8)[None, :], (8, 8 * F_out))

    grid = (S, M // _M_T)
    out_rows = pl.pallas_call(
        _tk_kernel,
        grid=grid,
        in_specs=[
            pl.BlockSpec((C + 1, 8, lanes), lambda s, t: (0, 0, 0)),
            pl.BlockSpec((3, 256, 512), lambda s, t: (0, 0, 0)),
            pl.BlockSpec((8, 512), lambda s, t: (0, 0)),
            pl.BlockSpec((512, 256), lambda s, t: (0, 0)),
            pl.BlockSpec((8, 256), lambda s, t: (0, 0)),
            pl.BlockSpec((1, 3, _M_T), lambda s, t: (s, 0, t),
                         memory_space=pltpu.SMEM),
            pl.BlockSpec((1, 3, _M_T), lambda s, t: (s, 0, t),
                         memory_space=pltpu.SMEM),
        ],
        out_specs=pl.BlockSpec((C, 8, lanes), lambda s, t: (0, 0, 0)),
        out_shape=jax.ShapeDtypeStruct((C, 8, lanes), jnp.float32),
        scratch_shapes=[
            pltpu.VMEM((3, _M_T, 8, lanes), jnp.bfloat16),
            pltpu.VMEM((_M_T, 8, lanes), jnp.float32),
        ],
        compiler_params=pltpu.CompilerParams(
            dimension_semantics=("arbitrary", "arbitrary"),
            vmem_limit_bytes=100 * 1024 * 1024),
    )(inp_rows, w1b, b1b, w2b, b2b, src_all, dst_all)

    return jnp.transpose(out_rows.reshape(C, R, F_out), (2, 1, 0))


# speedup vs baseline: 1.4052x; 1.1221x over previous
import functools

import jax
import jax.numpy as jnp
from jax import lax
from jax.experimental import pallas as pl
from jax.experimental.pallas import tpu as pltpu
from jax.experimental.pallas import tpu_sc as plsc

_M_T = 1024
_UNROLL = 16
_QW = 4


def _sc_index_kernel(indt_hbm, mix_hbm, src_hbm, dst_hbm,
                     keys_v, vals_v, lut_v, mix_v, src_v, dst_v):
    n_pairs = indt_hbm.shape[0]
    n_ind = indt_hbm.shape[2]
    m_tot = mix_hbm.shape[1]
    c_tot = lut_v.shape[0]
    nc = plsc.get_sparse_core_info().num_cores
    wid = lax.axis_index("s") * nc + lax.axis_index("c")

    @pl.when(wid < n_pairs)
    def _work():
        p = wid
        pltpu.sync_copy(indt_hbm.at[p, 0], keys_v.at[pl.ds(0, n_ind)])
        pltpu.sync_copy(indt_hbm.at[p, 1], vals_v)
        pltpu.sync_copy(mix_hbm.at[p], mix_v)
        keys_v[pl.ds(n_ind, 16)] = jnp.full((16,), -1, jnp.int32)

        cvec = jnp.full((16,), c_tot, jnp.int32)

        def init_body(j, carry):
            lut_v[pl.ds(j * 16, 16)] = cvec
            return carry

        lax.fori_loop(0, c_tot // 16, init_body, 0)

        def build_body(j, carry):
            k16 = keys_v[pl.ds(j * 16, 16)]
            v16 = vals_v[pl.ds(j * 16, 16)]
            dup = k16 != k16
            for sh in range(1, 16):
                dup = jnp.logical_or(
                    dup, keys_v[pl.ds(j * 16 + sh, 16)] == k16)
            plsc.store_scatter(lut_v, [k16], v16,
                               mask=jnp.logical_not(dup))
            return carry

        lax.fori_loop(0, n_ind // 16, build_body, 0)

        def gather_body(j, carry):
            idx = mix_v[pl.ds(j * 16, 16)]
            src_v[pl.ds(j * 16, 16)] = plsc.load_gather(lut_v, [idx])
            dst_v[pl.ds(j * 16, 16)] = plsc.load_gather(vals_v, [idx])
            return carry

        lax.fori_loop(0, m_tot // 16, gather_body, 0)
        pltpu.sync_copy(src_v, src_hbm.at[p])
        pltpu.sync_copy(dst_v, dst_hbm.at[p])


def _tk_kernel(inp_ref, w1b_ref, b1_ref, w2b_ref, b2_ref, src_ref, dst_ref,
               out_ref, g_ref, d_ref):
    s = pl.program_id(0)
    t = pl.program_id(1)

    @pl.when((s == 0) & (t == 0))
    def _zero_out():
        out_ref[...] = jnp.zeros(out_ref.shape, out_ref.dtype)

    def gather_body(k, carry):
        for u in range(_UNROLL):
            m = k * _UNROLL + u
            for i in range(3):
                g_ref[i, m] = inp_ref[src_ref[0, i, m]]
        return carry

    lax.fori_loop(0, _M_T // _UNROLL, gather_body, 0)

    x0 = g_ref[0].reshape(_M_T * 8, 256)
    x1 = g_ref[1].reshape(_M_T * 8, 256)
    x2 = g_ref[2].reshape(_M_T * 8, 256)
    z = (jnp.dot(x0, w1b_ref[0], preferred_element_type=jnp.float32)
         + jnp.dot(x1, w1b_ref[1], preferred_element_type=jnp.float32)
         + jnp.dot(x2, w1b_ref[2], preferred_element_type=jnp.float32))
    z = z + b1_ref[0:1, :]
    a = jnp.maximum(z, 0.0).astype(jnp.bfloat16)
    d = jnp.dot(a, w2b_ref[...], preferred_element_type=jnp.float32)
    d = d + b2_ref[0:1, :]
    d_ref[...] = d.reshape(_M_T, 8, 256)

    def scatter_body(k, carry):
        for u in range(_UNROLL):
            m = k * _UNROLL + u
            dval = d_ref[m]
            for i in range(3):
                c = dst_ref[0, i, m]
                out_ref[c] = jnp.maximum(out_ref[c], dval)
        return carry

    lax.fori_loop(0, _M_T // _UNROLL, scatter_body, 0)


@functools.partial(jax.jit, static_argnums=())
def kernel(input_tensor, ind0, ind1, ind2, mix_ind, w1, b1, w2, b2):
    F_in, R, C = input_tensor.shape
    F_out = w2.shape[0]
    S = ind0.shape[0]
    M = mix_ind.shape[2]
    RF = R * F_in
    lanes = RF // 8

    inp_rows = jnp.transpose(input_tensor, (2, 1, 0)).reshape(C, 8, lanes)
    inp_rows = jnp.concatenate(
        [inp_rows, jnp.zeros((1, 8, lanes), inp_rows.dtype)], axis=0)
    inp_rows = inp_rows.astype(jnp.bfloat16)

    inds = (ind0, ind1, ind2)
    indt_all = jnp.stack([jnp.transpose(inds[i][s], (1, 0))
                          for s in range(S) for i in range(3)])
    mix6 = mix_ind.reshape(3 * S, M)

    n_ind = ind0.shape[1]
    sc_mesh = plsc.VectorSubcoreMesh(core_axis_name="c", subcore_axis_name="s")
    src6, dst6 = pl.kernel(
        _sc_index_kernel,
        out_type=(jax.ShapeDtypeStruct((3 * S, M), jnp.int32),
                  jax.ShapeDtypeStruct((3 * S, M), jnp.int32)),
        mesh=sc_mesh,
        scratch_types=[
            pltpu.VMEM((n_ind + 16,), jnp.int32),
            pltpu.VMEM((n_ind,), jnp.int32),
            pltpu.VMEM((C,), jnp.int32),
            pltpu.VMEM((M,), jnp.int32),
            pltpu.VMEM((M,), jnp.int32),
            pltpu.VMEM((M,), jnp.int32),
        ],
        compiler_params=pltpu.CompilerParams(needs_layout_passes=False),
    )(indt_all, mix6)
    src_all = src6.reshape(S, 3, M)
    dst_all = dst6.reshape(S, 3, M)

    eye8 = jnp.eye(8, dtype=w1.dtype)
    w1b = jnp.stack([jnp.kron(eye8, w1[:, i * F_in:(i + 1) * F_in].T)
                     for i in range(3)]).astype(jnp.bfloat16)
    w2b = jnp.kron(eye8, w2.T).astype(jnp.bfloat16)
    b1b = jnp.broadcast_to(jnp.tile(b1, 8)[None, :], (8, 8 * w1.shape[0]))
    b2b = jnp.broadcast_to(jnp.tile(b2, 8)[None, :], (8, 8 * F_out))

    grid = (S, M // _M_T)
    out_rows = pl.pallas_call(
        _tk_kernel,
        grid=grid,
        in_specs=[
            pl.BlockSpec((C + 1, 8, lanes), lambda s, t: (0, 0, 0)),
            pl.BlockSpec((3, 256, 512), lambda s, t: (0, 0, 0)),
            pl.BlockSpec((8, 512), lambda s, t: (0, 0)),
            pl.BlockSpec((512, 256), lambda s, t: (0, 0)),
            pl.BlockSpec((8, 256), lambda s, t: (0, 0)),
            pl.BlockSpec((1, 3, _M_T), lambda s, t: (s, 0, t),
                         memory_space=pltpu.SMEM),
            pl.BlockSpec((1, 3, _M_T), lambda s, t: (s, 0, t),
                         memory_space=pltpu.SMEM),
        ],
        out_specs=pl.BlockSpec((C, 8, lanes), lambda s, t: (0, 0, 0)),
        out_shape=jax.ShapeDtypeStruct((C, 8, lanes), jnp.float32),
        scratch_shapes=[
            pltpu.VMEM((3, _M_T, 8, lanes), jnp.bfloat16),
            pltpu.VMEM((_M_T, 8, lanes), jnp.float32),
        ],
        compiler_params=pltpu.CompilerParams(
            dimension_semantics=("arbitrary", "arbitrary"),
            vmem_limit_bytes=100 * 1024 * 1024),
    )(inp_rows, w1b, b1b, w2b, b2b, src_all, dst_all)

    return jnp.transpose(out_rows.reshape(C, R, F_out), (2, 1, 0))


# speedup vs baseline: 1.4056x; 1.0003x over previous
import functools

import jax
import jax.numpy as jnp
from jax import lax
from jax.experimental import pallas as pl
from jax.experimental.pallas import tpu as pltpu
from jax.experimental.pallas import tpu_sc as plsc

_M_T = 1024
_UNROLL = 16


def _sc_index_kernel(indt_hbm, mix_hbm, src_hbm, dst_hbm,
                     keys_v, vals_v, lut_v, mix_v, src_v, dst_v):
    n_pairs = indt_hbm.shape[0]
    n_ind = indt_hbm.shape[2]
    m_tot = mix_hbm.shape[1]
    c_tot = lut_v.shape[0]
    nc = plsc.get_sparse_core_info().num_cores
    wid = lax.axis_index("s") * nc + lax.axis_index("c")

    @pl.when(wid < n_pairs)
    def _work():
        p = wid
        pltpu.sync_copy(indt_hbm.at[p, 0], keys_v.at[pl.ds(0, n_ind)])
        pltpu.sync_copy(indt_hbm.at[p, 1], vals_v)
        pltpu.sync_copy(mix_hbm.at[p], mix_v)
        keys_v[pl.ds(n_ind, 16)] = jnp.full((16,), -1, jnp.int32)

        cvec = jnp.full((16,), c_tot, jnp.int32)

        def init_body(j, carry):
            lut_v[pl.ds(j * 16, 16)] = cvec
            return carry

        lax.fori_loop(0, c_tot // 16, init_body, 0)

        def build_body(j, carry):
            k16 = keys_v[pl.ds(j * 16, 16)]
            v16 = vals_v[pl.ds(j * 16, 16)]
            dup = k16 != k16
            for sh in range(1, 16):
                dup = jnp.logical_or(
                    dup, keys_v[pl.ds(j * 16 + sh, 16)] == k16)
            plsc.store_scatter(lut_v, [k16], v16,
                               mask=jnp.logical_not(dup))
            return carry

        lax.fori_loop(0, n_ind // 16, build_body, 0)

        def gather_body(j, carry):
            idx = mix_v[pl.ds(j * 16, 16)]
            src_v[pl.ds(j * 16, 16)] = plsc.load_gather(lut_v, [idx])
            dst_v[pl.ds(j * 16, 16)] = plsc.load_gather(vals_v, [idx])
            return carry

        lax.fori_loop(0, m_tot // 16, gather_body, 0)
        pltpu.sync_copy(src_v, src_hbm.at[p])
        pltpu.sync_copy(dst_v, dst_hbm.at[p])


def _tk_kernel(inp_ref, w1b_ref, b1_ref, w2b_ref, b2_ref, src_ref, dst_ref,
               out_ref, g_ref, d_ref):
    s = pl.program_id(0)
    t = pl.program_id(1)

    @pl.when((s == 0) & (t == 0))
    def _zero_out():
        out_ref[...] = jnp.zeros(out_ref.shape, out_ref.dtype)

    def gather_body(k, carry):
        for u in range(_UNROLL):
            m = k * _UNROLL + u
            for i in range(3):
                g_ref[i, m] = inp_ref[src_ref[0, i, m]]
        return carry

    lax.fori_loop(0, _M_T // _UNROLL, gather_body, 0)

    x0 = g_ref[0].reshape(_M_T * 8, 256)
    x1 = g_ref[1].reshape(_M_T * 8, 256)
    x2 = g_ref[2].reshape(_M_T * 8, 256)
    z = (jnp.dot(x0, w1b_ref[0], preferred_element_type=jnp.float32)
         + jnp.dot(x1, w1b_ref[1], preferred_element_type=jnp.float32)
         + jnp.dot(x2, w1b_ref[2], preferred_element_type=jnp.float32))
    z = z + b1_ref[0:1, :]
    a = jnp.maximum(z, 0.0).astype(jnp.bfloat16)
    d = jnp.dot(a, w2b_ref[...], preferred_element_type=jnp.float32)
    d = d + b2_ref[0:1, :]
    d_ref[...] = d.reshape(_M_T, 8, 256)

    def scatter_body(k, carry):
        for u in range(_UNROLL):
            m = k * _UNROLL + u
            dval = d_ref[m]
            for i in range(3):
                c = dst_ref[0, i, m]
                out_ref[c] = jnp.maximum(out_ref[c], dval)
        return carry

    lax.fori_loop(0, _M_T // _UNROLL, scatter_body, 0)


@functools.partial(jax.jit, static_argnums=())
def kernel(input_tensor, ind0, ind1, ind2, mix_ind, w1, b1, w2, b2):
    F_in, R, C = input_tensor.shape
    F_out = w2.shape[0]
    S = ind0.shape[0]
    M = mix_ind.shape[2]
    RF = R * F_in
    lanes = RF // 8

    inp_rows = jnp.transpose(input_tensor, (2, 1, 0)).reshape(C, 8, lanes)
    inp_rows = jnp.concatenate(
        [inp_rows, jnp.zeros((1, 8, lanes), inp_rows.dtype)], axis=0)
    inp_rows = inp_rows.astype(jnp.bfloat16)

    inds = (ind0, ind1, ind2)
    indt_all = jnp.stack([jnp.transpose(inds[i][s], (1, 0))
                          for s in range(S) for i in range(3)])
    mix6 = mix_ind.reshape(3 * S, M)

    n_ind = ind0.shape[1]
    sc_mesh = plsc.VectorSubcoreMesh(core_axis_name="c", subcore_axis_name="s")
    src6, dst6 = pl.kernel(
        _sc_index_kernel,
        out_type=(jax.ShapeDtypeStruct((3 * S, M), jnp.int32),
                  jax.ShapeDtypeStruct((3 * S, M), jnp.int32)),
        mesh=sc_mesh,
        scratch_types=[
            pltpu.VMEM((n_ind + 16,), jnp.int32),
            pltpu.VMEM((n_ind,), jnp.int32),
            pltpu.VMEM((C,), jnp.int32),
            pltpu.VMEM((M,), jnp.int32),
            pltpu.VMEM((M,), jnp.int32),
            pltpu.VMEM((M,), jnp.int32),
        ],
        compiler_params=pltpu.CompilerParams(needs_layout_passes=False),
    )(indt_all, mix6)
    src_all = src6.reshape(S, 3, M)
    dst_all = dst6.reshape(S, 3, M)

    eye8 = jnp.eye(8, dtype=w1.dtype)
    w1b = jnp.stack([jnp.kron(eye8, w1[:, i * F_in:(i + 1) * F_in].T)
                     for i in range(3)]).astype(jnp.bfloat16)
    w2b = jnp.kron(eye8, w2.T).astype(jnp.bfloat16)
    b1b = jnp.broadcast_to(jnp.tile(b1, 8)[None, :], (8, 8 * w1.shape[0]))
    b2b = jnp.broadcast_to(jnp.tile(b2, 8)[None, :], (8, 8 * F_out))

    grid = (S, M // _M_T)
    out_rows = pl.pallas_call(
        _tk_kernel,
        grid=grid,
        in_specs=[
            pl.BlockSpec((C + 1, 8, lanes), lambda s, t: (0, 0, 0)),
            pl.BlockSpec((3, 256, 512), lambda s, t: (0, 0, 0)),
            pl.BlockSpec((8, 512), lambda s, t: (0, 0)),
            pl.BlockSpec((512, 256), lambda s, t: (0, 0)),
            pl.BlockSpec((8, 256), lambda s, t: (0, 0)),
            pl.BlockSpec((1, 3, _M_T), lambda s, t: (s, 0, t),
                         memory_space=pltpu.SMEM),
            pl.BlockSpec((1, 3, _M_T), lambda s, t: (s, 0, t),
                         memory_space=pltpu.SMEM),
        ],
        out_specs=pl.BlockSpec((C, 8, lanes), lambda s, t: (0, 0, 0)),
        out_shape=jax.ShapeDtypeStruct((C, 8, lanes), jnp.float32),
        scratch_shapes=[
            pltpu.VMEM((3, _M_T, 8, lanes), jnp.bfloat16),
            pltpu.VMEM((_M_T, 8, lanes), jnp.float32),
        ],
        compiler_params=pltpu.CompilerParams(
            dimension_semantics=("arbitrary", "arbitrary"),
            vmem_limit_bytes=100 * 1024 * 1024),
    )(inp_rows, w1b, b1b, w2b, b2b, src_all, dst_all)

    return jnp.transpose(out_rows.reshape(C, R, F_out), (2, 1, 0))
